# baseline (device time: 27388 ns/iter reference)
import jax
import jax.numpy as jnp
from jax import lax
from jax.experimental import pallas as pl
from jax.experimental.pallas import tpu as pltpu

C = 8


def kernel(x, dy):
    m, d = x.shape
    _, f = dy.shape
    fh = f // 2
    dh = d // 2
    cw = fh // C
    L = C - 1
    loL = L * cw

    tdims = (((0,), (0,)), ((), ()))

    def body(x_ref, dy_ref, out_ref, p_ref, xr_ref, r_ref, yr_ref, dr_ref,
             sx, rx, sy, ry, sd, rd):
        px = lax.axis_index("x")
        py = lax.axis_index("y")

        barrier = pltpu.get_barrier_semaphore()
        pl.semaphore_signal(barrier, inc=1, device_id=(1 - px, py),
                            device_id_type=pl.DeviceIdType.MESH)
        pl.semaphore_signal(barrier, inc=1, device_id=(px, 1 - py),
                            device_id_type=pl.DeviceIdType.MESH)
        pl.semaphore_signal(barrier, inc=1, device_id=(1 - px, 1 - py),
                            device_id_type=pl.DeviceIdType.MESH)
        pl.semaphore_wait(barrier, 3)

        def run(col0):
            oc0 = fh - col0
            x_rdmas = []
            y_rdmas = []

            def process(c):
                lo = c * cw
                x_rdmas[c].wait_recv()
                red = p_ref[pl.ds(px * dh, dh), lo:lo + cw] + xr_ref[:, lo:lo + cw]
                r_ref[:, lo:lo + cw] = red
                y_rdma = pltpu.make_async_remote_copy(
                    src_ref=r_ref.at[:, lo:lo + cw],
                    dst_ref=yr_ref.at[:, lo:lo + cw],
                    send_sem=sy.at[c],
                    recv_sem=ry.at[c],
                    device_id=(px, 1 - py),
                    device_id_type=pl.DeviceIdType.MESH,
                )
                y_rdma.start()
                y_rdmas.append(y_rdma)
                out_ref[:, col0 + lo:col0 + lo + cw] = red

            for c in range(C):
                lo = c * cw
                p_ref[:, lo:lo + cw] = lax.dot_general(
                    x_ref[...], dy_ref[:, col0 + lo:col0 + lo + cw], tdims,
                    preferred_element_type=jnp.float32)
                x_rdma = pltpu.make_async_remote_copy(
                    src_ref=p_ref.at[pl.ds((1 - px) * dh, dh), lo:lo + cw],
                    dst_ref=xr_ref.at[:, lo:lo + cw],
                    send_sem=sx.at[c],
                    recv_sem=rx.at[c],
                    device_id=(1 - px, py),
                    device_id_type=pl.DeviceIdType.MESH,
                )
                x_rdma.start()
                x_rdmas.append(x_rdma)
                if c == L:
                    y_raw = pltpu.make_async_remote_copy(
                        src_ref=p_ref.at[pl.ds(px * dh, dh), loL:loL + cw],
                        dst_ref=yr_ref.at[:, loL:loL + cw],
                        send_sem=sy.at[L],
                        recv_sem=ry.at[L],
                        device_id=(px, 1 - py),
                        device_id_type=pl.DeviceIdType.MESH,
                    )
                    y_raw.start()
                    y_rdmas.append(y_raw)
                    d_raw = pltpu.make_async_remote_copy(
                        src_ref=p_ref.at[pl.ds((1 - px) * dh, dh), loL:loL + cw],
                        dst_ref=dr_ref,
                        send_sem=sd,
                        recv_sem=rd,
                        device_id=(1 - px, 1 - py),
                        device_id_type=pl.DeviceIdType.MESH,
                    )
                    d_raw.start()
                    y_rdmas.append(d_raw)
                if c >= 1:
                    process(c - 1)
            x_rdmas[L].wait_recv()
            out_ref[:, col0 + loL:col0 + loL + cw] = (
                p_ref[pl.ds(px * dh, dh), loL:loL + cw] + xr_ref[:, loL:loL + cw])

            for c in range(C - 1):
                lo = c * cw
                yin = pltpu.make_async_remote_copy(
                    src_ref=r_ref.at[:, lo:lo + cw],
                    dst_ref=yr_ref.at[:, lo:lo + cw],
                    send_sem=sy.at[c],
                    recv_sem=ry.at[c],
                    device_id=(px, 1 - py),
                    device_id_type=pl.DeviceIdType.MESH,
                )
                yin.wait_recv()
                out_ref[:, oc0 + lo:oc0 + lo + cw] = yr_ref[:, lo:lo + cw]

            y_rawin = pltpu.make_async_remote_copy(
                src_ref=r_ref.at[:, loL:loL + cw],
                dst_ref=yr_ref.at[:, loL:loL + cw],
                send_sem=sy.at[L],
                recv_sem=ry.at[L],
                device_id=(px, 1 - py),
                device_id_type=pl.DeviceIdType.MESH,
            )
            y_rawin.wait_recv()
            d_rawin = pltpu.make_async_remote_copy(
                src_ref=r_ref.at[:, loL:loL + cw],
                dst_ref=dr_ref,
                send_sem=sd,
                recv_sem=rd,
                device_id=(1 - px, 1 - py),
                device_id_type=pl.DeviceIdType.MESH,
            )
            d_rawin.wait_recv()
            out_ref[:, oc0 + loL:oc0 + loL + cw] = (
                yr_ref[:, loL:loL + cw] + dr_ref[...])

            for rr in x_rdmas:
                rr.wait_send()
            for rr in y_rdmas:
                rr.wait_send()

        pl.when(py == 0)(lambda: run(0))
        pl.when(py == 1)(lambda: run(fh))

    return pl.pallas_call(
        body,
        out_shape=jax.ShapeDtypeStruct((dh, f), jnp.float32),
        in_specs=[pl.BlockSpec(memory_space=pltpu.VMEM),
                  pl.BlockSpec(memory_space=pltpu.VMEM)],
        out_specs=pl.BlockSpec(memory_space=pltpu.VMEM),
        scratch_shapes=[
            pltpu.VMEM((d, fh), jnp.float32),
            pltpu.VMEM((dh, fh), jnp.float32),
            pltpu.VMEM((dh, fh), jnp.float32),
            pltpu.VMEM((dh, fh), jnp.float32),
            pltpu.VMEM((dh, cw), jnp.float32),
            pltpu.SemaphoreType.DMA((C,)),
            pltpu.SemaphoreType.DMA((C,)),
            pltpu.SemaphoreType.DMA((C,)),
            pltpu.SemaphoreType.DMA((C,)),
            pltpu.SemaphoreType.DMA,
            pltpu.SemaphoreType.DMA,
        ],
        compiler_params=pltpu.CompilerParams(collective_id=0),
    )(x, dy)


# device time: 24241 ns/iter; 1.1298x vs baseline; 1.1298x over previous
import jax
import jax.numpy as jnp
from jax import lax
from jax.experimental import pallas as pl
from jax.experimental.pallas import tpu as pltpu

W = [256, 256, 256, 128, 128]
C = len(W)
OFF = [sum(W[:i]) for i in range(C)]
EW = 128
FW = sum(W) - EW


def kernel(x, dy):
    m, d = x.shape
    _, f = dy.shape
    fh = f // 2
    dh = d // 2
    assert sum(W) == fh

    FWD = [c for c in range(C) if OFF[c] + W[c] <= FW]

    tdims = (((0,), (0,)), ((), ()))

    def body(x_ref, dy_ref, out_ref, p_ref, xr_ref, r_ref, yr_ref,
             pe_ref, xoc_ref, sx, rx, sy, ry, sxe, rxe):
        px = lax.axis_index("x")
        py = lax.axis_index("y")

        barrier = pltpu.get_barrier_semaphore()
        pl.semaphore_signal(barrier, inc=1, device_id=(1 - px, py),
                            device_id_type=pl.DeviceIdType.MESH)
        pl.semaphore_signal(barrier, inc=1, device_id=(px, 1 - py),
                            device_id_type=pl.DeviceIdType.MESH)
        pl.semaphore_wait(barrier, 2)

        def run(col0):
            oc0 = fh - col0
            x_rdmas = []
            y_rdmas = []

            def process(c):
                lo, w = OFF[c], W[c]
                x_rdmas[c].wait_recv()
                red = p_ref[pl.ds(px * dh, dh), lo:lo + w] + xr_ref[:, lo:lo + w]
                if c in FWD:
                    r_ref[:, lo:lo + w] = red
                    y_rdma = pltpu.make_async_remote_copy(
                        src_ref=r_ref.at[:, lo:lo + w],
                        dst_ref=yr_ref.at[:, lo:lo + w],
                        send_sem=sy.at[c],
                        recv_sem=ry.at[c],
                        device_id=(px, 1 - py),
                        device_id_type=pl.DeviceIdType.MESH,
                    )
                    y_rdma.start()
                    y_rdmas.append(y_rdma)
                out_ref[:, col0 + lo:col0 + lo + w] = red

            for c in range(C):
                lo, w = OFF[c], W[c]
                p_ref[:, lo:lo + w] = lax.dot_general(
                    x_ref[...], dy_ref[:, col0 + lo:col0 + lo + w], tdims,
                    preferred_element_type=jnp.float32)
                x_rdma = pltpu.make_async_remote_copy(
                    src_ref=p_ref.at[pl.ds((1 - px) * dh, dh), lo:lo + w],
                    dst_ref=xr_ref.at[:, lo:lo + w],
                    send_sem=sx.at[c],
                    recv_sem=rx.at[c],
                    device_id=(1 - px, py),
                    device_id_type=pl.DeviceIdType.MESH,
                )
                x_rdma.start()
                x_rdmas.append(x_rdma)
                if c >= 1:
                    process(c - 1)

            pe_ref[...] = lax.dot_general(
                x_ref[...], dy_ref[:, oc0 + FW:oc0 + FW + EW], tdims,
                preferred_element_type=jnp.float32)
            pex = pltpu.make_async_remote_copy(
                src_ref=pe_ref.at[pl.ds((1 - px) * dh, dh), :],
                dst_ref=xoc_ref,
                send_sem=sxe,
                recv_sem=rxe,
                device_id=(1 - px, py),
                device_id_type=pl.DeviceIdType.MESH,
            )
            pex.start()
            process(C - 1)

            for c in FWD:
                lo, w = OFF[c], W[c]
                yin = pltpu.make_async_remote_copy(
                    src_ref=r_ref.at[:, lo:lo + w],
                    dst_ref=yr_ref.at[:, lo:lo + w],
                    send_sem=sy.at[c],
                    recv_sem=ry.at[c],
                    device_id=(px, 1 - py),
                    device_id_type=pl.DeviceIdType.MESH,
                )
                yin.wait_recv()
                out_ref[:, oc0 + lo:oc0 + lo + w] = yr_ref[:, lo:lo + w]

            pex.wait_recv()
            out_ref[:, oc0 + FW:oc0 + FW + EW] = (
                pe_ref[pl.ds(px * dh, dh), :] + xoc_ref[...])

            pex.wait_send()
            for rr in x_rdmas:
                rr.wait_send()
            for rr in y_rdmas:
                rr.wait_send()

        pl.when(py == 0)(lambda: run(0))
        pl.when(py == 1)(lambda: run(fh))

    return pl.pallas_call(
        body,
        out_shape=jax.ShapeDtypeStruct((dh, f), jnp.float32),
        in_specs=[pl.BlockSpec(memory_space=pltpu.VMEM),
                  pl.BlockSpec(memory_space=pltpu.VMEM)],
        out_specs=pl.BlockSpec(memory_space=pltpu.VMEM),
        scratch_shapes=[
            pltpu.VMEM((d, fh), jnp.float32),
            pltpu.VMEM((dh, fh), jnp.float32),
            pltpu.VMEM((dh, fh), jnp.float32),
            pltpu.VMEM((dh, fh), jnp.float32),
            pltpu.VMEM((d, EW), jnp.float32),
            pltpu.VMEM((dh, EW), jnp.float32),
            pltpu.SemaphoreType.DMA((C,)),
            pltpu.SemaphoreType.DMA((C,)),
            pltpu.SemaphoreType.DMA((C,)),
            pltpu.SemaphoreType.DMA((C,)),
            pltpu.SemaphoreType.DMA,
            pltpu.SemaphoreType.DMA,
        ],
        compiler_params=pltpu.CompilerParams(collective_id=0),
    )(x, dy)
